# Initial kernel scaffold; baseline (speedup 1.0000x reference)
import functools

import jax
import jax.numpy as jnp
from jax import lax
from jax.experimental import pallas as pl
from jax.experimental.pallas import tpu as pltpu
from jax.experimental.pallas import tpu_sc as plsc

N = 10000
E = 320000
D = 128
NC = 2
NS = 16
NW = NC * NS
EPW = E // NW
CHUNK = 80
NCHUNK = EPW // CHUNK
RPT = N // NS
DEGW = 16

_MESH = plsc.VectorSubcoreMesh(core_axis_name="c", subcore_axis_name="s")
_F32 = jnp.float32
_HIGH = lax.Precision.HIGHEST


def _sc_common(xl_hbm, src_v, dst_v, rows_v, acc_sp, sem, deg_body=None):
    @pl.loop(0, NCHUNK)
    def _(j):
        pltpu.async_copy(xl_hbm.at[src_v.at[j]], rows_v, sem).wait()
        pltpu.sync_copy(rows_v, acc_sp.at[dst_v.at[j]], add=True)
        if deg_body is not None:
            deg_body(j)


@functools.partial(
    pl.kernel,
    out_type=(jax.ShapeDtypeStruct((NC, N, D), _F32),
              jax.ShapeDtypeStruct((NC, N, DEGW), _F32)),
    mesh=_MESH,
    scratch_types=[
        pltpu.VMEM((NCHUNK, CHUNK), jnp.int32),
        pltpu.VMEM((NCHUNK, CHUNK), jnp.int32),
        pltpu.VMEM((CHUNK, D), _F32),
        pltpu.VMEM((CHUNK, DEGW), _F32),
        pltpu.VMEM_SHARED((N, D), _F32),
        pltpu.VMEM_SHARED((N, DEGW), _F32),
        pltpu.SemaphoreType.DMA,
    ],
)
def _sc_agg_deg(xl_hbm, src_hbm, dst_hbm, ones_hbm, zacc_hbm, zdeg_hbm,
                acc_out, deg_out,
                src_v, dst_v, rows_v, ones_v, acc_sp, deg_sp, sem):
    c = lax.axis_index("c")
    s = lax.axis_index("s")
    gid = c * NS + s
    pltpu.sync_copy(zacc_hbm.at[pl.ds(s * RPT, RPT)], acc_sp.at[pl.ds(s * RPT, RPT)])
    pltpu.sync_copy(zdeg_hbm.at[pl.ds(s * RPT, RPT)], deg_sp.at[pl.ds(s * RPT, RPT)])
    pltpu.sync_copy(src_hbm.at[gid], src_v)
    pltpu.sync_copy(dst_hbm.at[gid], dst_v)
    pltpu.sync_copy(ones_hbm, ones_v)
    plsc.subcore_barrier()

    def deg_body(j):
        pltpu.sync_copy(ones_v, deg_sp.at[dst_v.at[j]], add=True)

    _sc_common(xl_hbm, src_v, dst_v, rows_v, acc_sp, sem, deg_body)
    plsc.subcore_barrier()
    pltpu.sync_copy(acc_sp.at[pl.ds(s * RPT, RPT)],
                    acc_out.at[c, pl.ds(s * RPT, RPT)])
    pltpu.sync_copy(deg_sp.at[pl.ds(s * RPT, RPT)],
                    deg_out.at[c, pl.ds(s * RPT, RPT)])


@functools.partial(
    pl.kernel,
    out_type=jax.ShapeDtypeStruct((NC, N, D), _F32),
    mesh=_MESH,
    scratch_types=[
        pltpu.VMEM((NCHUNK, CHUNK), jnp.int32),
        pltpu.VMEM((NCHUNK, CHUNK), jnp.int32),
        pltpu.VMEM((CHUNK, D), _F32),
        pltpu.VMEM_SHARED((N, D), _F32),
        pltpu.SemaphoreType.DMA,
    ],
)
def _sc_agg(xl_hbm, src_hbm, dst_hbm, zacc_hbm,
            acc_out,
            src_v, dst_v, rows_v, acc_sp, sem):
    c = lax.axis_index("c")
    s = lax.axis_index("s")
    gid = c * NS + s
    pltpu.sync_copy(zacc_hbm.at[pl.ds(s * RPT, RPT)], acc_sp.at[pl.ds(s * RPT, RPT)])
    pltpu.sync_copy(src_hbm.at[gid], src_v)
    pltpu.sync_copy(dst_hbm.at[gid], dst_v)
    plsc.subcore_barrier()
    _sc_common(xl_hbm, src_v, dst_v, rows_v, acc_sp, sem)
    plsc.subcore_barrier()
    pltpu.sync_copy(acc_sp.at[pl.ds(s * RPT, RPT)],
                    acc_out.at[c, pl.ds(s * RPT, RPT)])


_RB = 2000


def _tc_pre(x, W1_l, W1_r, b1):
    def body(x_ref, wl_ref, wr_ref, b_ref, xl_ref, xr_ref):
        xb = x_ref[...]
        xl_ref[...] = jnp.dot(xb, wl_ref[...], preferred_element_type=_F32,
                              precision=_HIGH)
        xr_ref[...] = jnp.dot(xb, wr_ref[...], preferred_element_type=_F32,
                              precision=_HIGH) + b_ref[...]
    return pl.pallas_call(
        body,
        grid=(N // _RB,),
        in_specs=[pl.BlockSpec((_RB, D), lambda i: (i, 0)),
                  pl.BlockSpec((D, D), lambda i: (0, 0)),
                  pl.BlockSpec((D, D), lambda i: (0, 0)),
                  pl.BlockSpec((1, D), lambda i: (0, 0))],
        out_specs=[pl.BlockSpec((_RB, D), lambda i: (i, 0))] * 2,
        out_shape=[jax.ShapeDtypeStruct((N, D), _F32)] * 2,
    )(x, W1_l, W1_r, b1)


def _tc_mid(acc1, deg, xr, W2_l, W2_r, b2):
    def body(acc_ref, deg_ref, xr_ref, wl_ref, wr_ref, b_ref, hl_ref, hr_ref):
        a = acc_ref[0] + acc_ref[1]
        dcnt = deg_ref[0, :, 0:1] + deg_ref[1, :, 0:1]
        h = jnp.maximum(a / jnp.maximum(dcnt, 1.0) + xr_ref[...], 0.0)
        hl_ref[...] = jnp.dot(h, wl_ref[...], preferred_element_type=_F32,
                              precision=_HIGH)
        hr_ref[...] = jnp.dot(h, wr_ref[...], preferred_element_type=_F32,
                              precision=_HIGH) + b_ref[...]
    return pl.pallas_call(
        body,
        grid=(N // _RB,),
        in_specs=[pl.BlockSpec((NC, _RB, D), lambda i: (0, i, 0)),
                  pl.BlockSpec((NC, _RB, DEGW), lambda i: (0, i, 0)),
                  pl.BlockSpec((_RB, D), lambda i: (i, 0)),
                  pl.BlockSpec((D, D), lambda i: (0, 0)),
                  pl.BlockSpec((D, D), lambda i: (0, 0)),
                  pl.BlockSpec((1, D), lambda i: (0, 0))],
        out_specs=[pl.BlockSpec((_RB, D), lambda i: (i, 0))] * 2,
        out_shape=[jax.ShapeDtypeStruct((N, D), _F32)] * 2,
    )(acc1, deg, xr, W2_l, W2_r, b2)


def _tc_post(acc2, deg, hr):
    def body(acc_ref, deg_ref, hr_ref, o_ref):
        a = acc_ref[0] + acc_ref[1]
        dcnt = deg_ref[0, :, 0:1] + deg_ref[1, :, 0:1]
        o_ref[...] = a / jnp.maximum(dcnt, 1.0) + hr_ref[...]
    return pl.pallas_call(
        body,
        grid=(N // _RB,),
        in_specs=[pl.BlockSpec((NC, _RB, D), lambda i: (0, i, 0)),
                  pl.BlockSpec((NC, _RB, DEGW), lambda i: (0, i, 0)),
                  pl.BlockSpec((_RB, D), lambda i: (i, 0))],
        out_specs=pl.BlockSpec((_RB, D), lambda i: (i, 0)),
        out_shape=jax.ShapeDtypeStruct((N, D), _F32),
    )(acc2, deg, hr)


def kernel(x, edge_index, W1_l, b1_l, W1_r, W2_l, b2_l, W2_r):
    src = edge_index[0].astype(jnp.int32).reshape(NW, NCHUNK, CHUNK)
    dst = edge_index[1].astype(jnp.int32).reshape(NW, NCHUNK, CHUNK)
    ones = jnp.zeros((CHUNK, DEGW), _F32).at[:, 0].set(1.0)
    zacc = jnp.zeros((N, D), _F32)
    zdeg = jnp.zeros((N, DEGW), _F32)
    b1 = b1_l.reshape(1, D)
    b2 = b2_l.reshape(1, D)

    xl, xr = _tc_pre(x, W1_l, W1_r, b1)
    acc1, deg = _sc_agg_deg(xl, src, dst, ones, zacc, zdeg)
    hl, hr = _tc_mid(acc1, deg, xr, W2_l, W2_r, b2)
    acc2 = _sc_agg(hl, src, dst, zacc)
    return _tc_post(acc2, deg, hr)


# speedup vs baseline: 5.5262x; 5.5262x over previous
import functools

import jax
import jax.numpy as jnp
from jax import lax
from jax.experimental import pallas as pl
from jax.experimental.pallas import tpu as pltpu
from jax.experimental.pallas import tpu_sc as plsc

N = 10000
E = 320000
D = 128
DH = D // 2
NC = 2
NS = 16
EPT = E // NS
CHUNK = 80
NCHUNK = EPT // CHUNK
NP = 10240
RPT = NP // NS
DEGW = 16

_MESH = plsc.VectorSubcoreMesh(core_axis_name="c", subcore_axis_name="s")
_F32 = jnp.float32
_HIGH = lax.Precision.HIGHEST


@functools.partial(
    pl.kernel,
    out_type=(jax.ShapeDtypeStruct((NC, NP, DH), _F32),
              jax.ShapeDtypeStruct((NP, DEGW), _F32)),
    mesh=_MESH,
    compiler_params=pltpu.CompilerParams(use_tc_tiling_on_sc=False),
    scratch_types=[
        pltpu.VMEM((NCHUNK, CHUNK), jnp.int32),
        pltpu.VMEM((NCHUNK, CHUNK), jnp.int32),
        pltpu.VMEM((CHUNK, DH), _F32),
        pltpu.VMEM((CHUNK, DEGW), _F32),
        pltpu.VMEM_SHARED((NP, DH), _F32),
        pltpu.VMEM_SHARED((NP, DEGW), _F32),
        pltpu.SemaphoreType.DMA,
    ],
)
def _sc_agg_deg(xl_hbm, src_hbm, dst_hbm, ones_hbm, zacc_hbm, zdeg_hbm,
                acc_out, deg_out,
                src_v, dst_v, rows_v, ones_v, acc_sp, deg_sp, sem):
    c = lax.axis_index("c")
    s = lax.axis_index("s")
    pltpu.sync_copy(zacc_hbm.at[pl.ds(s * RPT, RPT)], acc_sp.at[pl.ds(s * RPT, RPT)])
    pltpu.sync_copy(src_hbm.at[s], src_v)
    pltpu.sync_copy(dst_hbm.at[s], dst_v)

    @pl.when(c == 0)
    def _():
        pltpu.sync_copy(zdeg_hbm.at[pl.ds(s * RPT, RPT)],
                        deg_sp.at[pl.ds(s * RPT, RPT)])
        pltpu.sync_copy(ones_hbm, ones_v)

    plsc.subcore_barrier()
    my_half = xl_hbm.at[c]

    @pl.loop(0, NCHUNK)
    def _(j):
        pltpu.async_copy(my_half.at[src_v.at[j]], rows_v, sem).wait()
        pltpu.sync_copy(rows_v, acc_sp.at[dst_v.at[j]], add=True)

        @pl.when(c == 0)
        def _():
            pltpu.sync_copy(ones_v, deg_sp.at[dst_v.at[j]], add=True)

    plsc.subcore_barrier()
    pltpu.sync_copy(acc_sp.at[pl.ds(s * RPT, RPT)],
                    acc_out.at[c, pl.ds(s * RPT, RPT)])

    @pl.when(c == 0)
    def _():
        pltpu.sync_copy(deg_sp.at[pl.ds(s * RPT, RPT)],
                        deg_out.at[pl.ds(s * RPT, RPT)])


@functools.partial(
    pl.kernel,
    out_type=jax.ShapeDtypeStruct((NC, NP, DH), _F32),
    mesh=_MESH,
    compiler_params=pltpu.CompilerParams(use_tc_tiling_on_sc=False),
    scratch_types=[
        pltpu.VMEM((NCHUNK, CHUNK), jnp.int32),
        pltpu.VMEM((NCHUNK, CHUNK), jnp.int32),
        pltpu.VMEM((CHUNK, DH), _F32),
        pltpu.VMEM_SHARED((NP, DH), _F32),
        pltpu.SemaphoreType.DMA,
    ],
)
def _sc_agg(xl_hbm, src_hbm, dst_hbm, zacc_hbm,
            acc_out,
            src_v, dst_v, rows_v, acc_sp, sem):
    c = lax.axis_index("c")
    s = lax.axis_index("s")
    pltpu.sync_copy(zacc_hbm.at[pl.ds(s * RPT, RPT)], acc_sp.at[pl.ds(s * RPT, RPT)])
    pltpu.sync_copy(src_hbm.at[s], src_v)
    pltpu.sync_copy(dst_hbm.at[s], dst_v)
    plsc.subcore_barrier()
    my_half = xl_hbm.at[c]

    @pl.loop(0, NCHUNK)
    def _(j):
        pltpu.async_copy(my_half.at[src_v.at[j]], rows_v, sem).wait()
        pltpu.sync_copy(rows_v, acc_sp.at[dst_v.at[j]], add=True)

    plsc.subcore_barrier()
    pltpu.sync_copy(acc_sp.at[pl.ds(s * RPT, RPT)],
                    acc_out.at[c, pl.ds(s * RPT, RPT)])


_RB = 2000


def _tc_pre(x, W1_l, W1_r, b1):
    def body(x_ref, wl_ref, wr_ref, b_ref, xl_ref, xr_ref):
        xb = x_ref[...]
        xl = jnp.dot(xb, wl_ref[...], preferred_element_type=_F32,
                     precision=_HIGH)
        xl_ref[0, ...] = xl[:, :DH]
        xl_ref[1, ...] = xl[:, DH:]
        xr_ref[...] = jnp.dot(xb, wr_ref[...], preferred_element_type=_F32,
                              precision=_HIGH) + b_ref[...]
    return pl.pallas_call(
        body,
        grid=(N // _RB,),
        in_specs=[pl.BlockSpec((_RB, D), lambda i: (i, 0)),
                  pl.BlockSpec((D, D), lambda i: (0, 0)),
                  pl.BlockSpec((D, D), lambda i: (0, 0)),
                  pl.BlockSpec((1, D), lambda i: (0, 0))],
        out_specs=[pl.BlockSpec((NC, _RB, DH), lambda i: (0, i, 0)),
                   pl.BlockSpec((_RB, D), lambda i: (i, 0))],
        out_shape=[jax.ShapeDtypeStruct((NC, N, DH), _F32),
                   jax.ShapeDtypeStruct((N, D), _F32)],
    )(x, W1_l, W1_r, b1)


def _tc_mid(acc1, deg, xr, W2_l, W2_r, b2):
    def body(acc_ref, deg_ref, xr_ref, wl_ref, wr_ref, b_ref, hl_ref, hr_ref):
        a = jnp.concatenate([acc_ref[0], acc_ref[1]], axis=1)
        dcnt = deg_ref[:, 0:1]
        h = jnp.maximum(a / jnp.maximum(dcnt, 1.0) + xr_ref[...], 0.0)
        hl = jnp.dot(h, wl_ref[...], preferred_element_type=_F32,
                     precision=_HIGH)
        hl_ref[0, ...] = hl[:, :DH]
        hl_ref[1, ...] = hl[:, DH:]
        hr_ref[...] = jnp.dot(h, wr_ref[...], preferred_element_type=_F32,
                              precision=_HIGH) + b_ref[...]
    return pl.pallas_call(
        body,
        grid=(N // _RB,),
        in_specs=[pl.BlockSpec((NC, _RB, DH), lambda i: (0, i, 0)),
                  pl.BlockSpec((_RB, DEGW), lambda i: (i, 0)),
                  pl.BlockSpec((_RB, D), lambda i: (i, 0)),
                  pl.BlockSpec((D, D), lambda i: (0, 0)),
                  pl.BlockSpec((D, D), lambda i: (0, 0)),
                  pl.BlockSpec((1, D), lambda i: (0, 0))],
        out_specs=[pl.BlockSpec((NC, _RB, DH), lambda i: (0, i, 0)),
                   pl.BlockSpec((_RB, D), lambda i: (i, 0))],
        out_shape=[jax.ShapeDtypeStruct((NC, N, DH), _F32),
                   jax.ShapeDtypeStruct((N, D), _F32)],
    )(acc1, deg, xr, W2_l, W2_r, b2)


def _tc_post(acc2, deg, hr):
    def body(acc_ref, deg_ref, hr_ref, o_ref):
        a = jnp.concatenate([acc_ref[0], acc_ref[1]], axis=1)
        dcnt = deg_ref[:, 0:1]
        o_ref[...] = a / jnp.maximum(dcnt, 1.0) + hr_ref[...]
    return pl.pallas_call(
        body,
        grid=(N // _RB,),
        in_specs=[pl.BlockSpec((NC, _RB, DH), lambda i: (0, i, 0)),
                  pl.BlockSpec((_RB, DEGW), lambda i: (i, 0)),
                  pl.BlockSpec((_RB, D), lambda i: (i, 0))],
        out_specs=pl.BlockSpec((_RB, D), lambda i: (i, 0)),
        out_shape=jax.ShapeDtypeStruct((N, D), _F32),
    )(acc2, deg, hr)


def kernel(x, edge_index, W1_l, b1_l, W1_r, W2_l, b2_l, W2_r):
    src = edge_index[0].astype(jnp.int32).reshape(NS, NCHUNK, CHUNK)
    dst = edge_index[1].astype(jnp.int32).reshape(NS, NCHUNK, CHUNK)
    ones = jnp.zeros((CHUNK, DEGW), _F32).at[:, 0].set(1.0)
    zacc = jnp.zeros((NP, DH), _F32)
    zdeg = jnp.zeros((NP, DEGW), _F32)
    b1 = b1_l.reshape(1, D)
    b2 = b2_l.reshape(1, D)

    xl, xr = _tc_pre(x, W1_l, W1_r, b1)
    acc1, deg = _sc_agg_deg(xl, src, dst, ones, zacc, zdeg)
    hl, hr = _tc_mid(acc1, deg, xr, W2_l, W2_r, b2)
    acc2 = _sc_agg(hl, src, dst, zacc)
    return _tc_post(acc2, deg, hr)


# speedup vs baseline: 10.3345x; 1.8701x over previous
import jax
import jax.numpy as jnp
from jax import lax
from jax.experimental import pallas as pl
from jax.experimental.pallas import tpu as pltpu
from jax.experimental.pallas import tpu_sc as plsc

N = 10000
E = 320000
D = 128
DH = D // 2
NC = 2
NS = 16
EPT = E // NS
CHUNK = 80
NCHUNK = EPT // CHUNK
NP = 10240
RPT = NP // NS
DEGW = 16
NBUF = 3

_MESH = plsc.VectorSubcoreMesh(core_axis_name="c", subcore_axis_name="s")
_F32 = jnp.float32
_HIGH = lax.Precision.HIGHEST


def _build_sc(with_deg):
    out_type = [jax.ShapeDtypeStruct((NC, NP, DH), _F32)]
    scratch = (
        [pltpu.VMEM((NCHUNK, CHUNK), jnp.int32),
         pltpu.VMEM((NCHUNK, CHUNK), jnp.int32)]
        + [pltpu.VMEM((CHUNK, DH), _F32)] * NBUF
        + [pltpu.VMEM_SHARED((NP, DH), _F32)]
        + [pltpu.SemaphoreType.DMA] * (2 * NBUF)
    )
    if with_deg:
        out_type.append(jax.ShapeDtypeStruct((NC, NP, DEGW), _F32))
        scratch += [pltpu.VMEM((CHUNK, DEGW), _F32),
                    pltpu.VMEM_SHARED((NP, DEGW), _F32),
                    pltpu.SemaphoreType.DMA]

    def body(*refs):
        if with_deg:
            (xl_hbm, src_hbm, dst_hbm, ones_hbm, zacc_hbm, zdeg_hbm,
             acc_out, deg_out,
             src_v, dst_v, b0, b1, b2, acc_sp,
             g0, g1, g2, s0, s1, s2, ones_v, deg_sp, dsem) = refs
        else:
            (xl_hbm, src_hbm, dst_hbm, zacc_hbm,
             acc_out,
             src_v, dst_v, b0, b1, b2, acc_sp,
             g0, g1, g2, s0, s1, s2) = refs
        c = lax.axis_index("c")
        s = lax.axis_index("s")
        pltpu.sync_copy(zacc_hbm.at[pl.ds(s * RPT, RPT)],
                        acc_sp.at[pl.ds(s * RPT, RPT)])
        pltpu.sync_copy(src_hbm.at[s], src_v)
        pltpu.sync_copy(dst_hbm.at[s], dst_v)
        if with_deg:
            pltpu.sync_copy(zdeg_hbm.at[pl.ds(s * RPT, RPT)],
                            deg_sp.at[pl.ds(s * RPT, RPT)])
            pltpu.sync_copy(ones_hbm, ones_v)
        plsc.subcore_barrier()
        half = xl_hbm.at[c]
        bufs = (b0, b1, b2)
        gsem = (g0, g1, g2)
        ssem = (s0, s1, s2)

        def g_start(t, i):
            pltpu.async_copy(half.at[src_v.at[t]], bufs[i], gsem[i])

        def g_wait(i):
            pltpu.make_async_copy(half.at[src_v.at[0]], bufs[i], gsem[i]).wait()

        def s_start(t, i):
            pltpu.async_copy(bufs[i], acc_sp.at[dst_v.at[t]], ssem[i], add=True)

        def s_wait(i):
            pltpu.make_async_copy(bufs[i], acc_sp.at[dst_v.at[0]], ssem[i]).wait()

        if with_deg:
            base = c * (NCHUNK // 2)

            @pl.loop(0, NCHUNK // 2)
            def _(i):
                pltpu.async_copy(ones_v, deg_sp.at[dst_v.at[base + i]],
                                 dsem, add=True)

        def slot(t, i, has_swait=True, has_gnext=True):
            iw = (i + 1) % NBUF
            if has_swait:
                s_wait(iw)
            if has_gnext:
                g_start(t + 1, iw)
            g_wait(i)
            s_start(t, i)

        g_start(0, 0)
        slot(0, 0, has_swait=False)
        slot(1, 1, has_swait=False)
        slot(2, 2)

        @pl.loop(0, (NCHUNK - 7) // NBUF)
        def _(g):
            t0 = NBUF * g + 3
            slot(t0, 0)
            slot(t0 + 1, 1)
            slot(t0 + 2, 2)

        slot(NCHUNK - 4, (NCHUNK - 4) % NBUF)
        slot(NCHUNK - 3, (NCHUNK - 3) % NBUF)
        slot(NCHUNK - 2, (NCHUNK - 2) % NBUF)
        slot(NCHUNK - 1, (NCHUNK - 1) % NBUF, has_gnext=False)
        s_wait((NCHUNK - 2) % NBUF)
        s_wait((NCHUNK - 1) % NBUF)

        if with_deg:
            @pl.loop(0, NCHUNK // 2)
            def _(i):
                pltpu.make_async_copy(ones_v, deg_sp.at[dst_v.at[0]],
                                      dsem).wait()

        plsc.subcore_barrier()
        pltpu.sync_copy(acc_sp.at[pl.ds(s * RPT, RPT)],
                        acc_out.at[c, pl.ds(s * RPT, RPT)])
        if with_deg:
            pltpu.sync_copy(deg_sp.at[pl.ds(s * RPT, RPT)],
                            deg_out.at[c, pl.ds(s * RPT, RPT)])

    return pl.kernel(
        body,
        out_type=tuple(out_type) if with_deg else out_type[0],
        mesh=_MESH,
        compiler_params=pltpu.CompilerParams(use_tc_tiling_on_sc=False),
        scratch_types=scratch,
    )


_SC_AGG_DEG = _build_sc(True)
_SC_AGG = _build_sc(False)

_RB = 2000


def _tc_pre(x, W1_l, W1_r, b1):
    def body(x_ref, wl_ref, wr_ref, b_ref, xl_ref, xr_ref):
        xb = x_ref[...]
        xl = jnp.dot(xb, wl_ref[...], preferred_element_type=_F32,
                     precision=_HIGH)
        xl_ref[0, ...] = xl[:, :DH]
        xl_ref[1, ...] = xl[:, DH:]
        xr_ref[...] = jnp.dot(xb, wr_ref[...], preferred_element_type=_F32,
                              precision=_HIGH) + b_ref[...]
    return pl.pallas_call(
        body,
        grid=(N // _RB,),
        in_specs=[pl.BlockSpec((_RB, D), lambda i: (i, 0)),
                  pl.BlockSpec((D, D), lambda i: (0, 0)),
                  pl.BlockSpec((D, D), lambda i: (0, 0)),
                  pl.BlockSpec((1, D), lambda i: (0, 0))],
        out_specs=[pl.BlockSpec((NC, _RB, DH), lambda i: (0, i, 0)),
                   pl.BlockSpec((_RB, D), lambda i: (i, 0))],
        out_shape=[jax.ShapeDtypeStruct((NC, N, DH), _F32),
                   jax.ShapeDtypeStruct((N, D), _F32)],
    )(x, W1_l, W1_r, b1)


def _tc_mid(acc1, deg, xr, W2_l, W2_r, b2):
    def body(acc_ref, deg_ref, xr_ref, wl_ref, wr_ref, b_ref, hl_ref, hr_ref):
        a = jnp.concatenate([acc_ref[0], acc_ref[1]], axis=1)
        dcnt = deg_ref[0, :, 0:1] + deg_ref[1, :, 0:1]
        h = jnp.maximum(a / jnp.maximum(dcnt, 1.0) + xr_ref[...], 0.0)
        hl = jnp.dot(h, wl_ref[...], preferred_element_type=_F32,
                     precision=_HIGH)
        hl_ref[0, ...] = hl[:, :DH]
        hl_ref[1, ...] = hl[:, DH:]
        hr_ref[...] = jnp.dot(h, wr_ref[...], preferred_element_type=_F32,
                              precision=_HIGH) + b_ref[...]
    return pl.pallas_call(
        body,
        grid=(N // _RB,),
        in_specs=[pl.BlockSpec((NC, _RB, DH), lambda i: (0, i, 0)),
                  pl.BlockSpec((NC, _RB, DEGW), lambda i: (0, i, 0)),
                  pl.BlockSpec((_RB, D), lambda i: (i, 0)),
                  pl.BlockSpec((D, D), lambda i: (0, 0)),
                  pl.BlockSpec((D, D), lambda i: (0, 0)),
                  pl.BlockSpec((1, D), lambda i: (0, 0))],
        out_specs=[pl.BlockSpec((NC, _RB, DH), lambda i: (0, i, 0)),
                   pl.BlockSpec((_RB, D), lambda i: (i, 0))],
        out_shape=[jax.ShapeDtypeStruct((NC, N, DH), _F32),
                   jax.ShapeDtypeStruct((N, D), _F32)],
    )(acc1, deg, xr, W2_l, W2_r, b2)


def _tc_post(acc2, deg, hr):
    def body(acc_ref, deg_ref, hr_ref, o_ref):
        a = jnp.concatenate([acc_ref[0], acc_ref[1]], axis=1)
        dcnt = deg_ref[0, :, 0:1] + deg_ref[1, :, 0:1]
        o_ref[...] = a / jnp.maximum(dcnt, 1.0) + hr_ref[...]
    return pl.pallas_call(
        body,
        grid=(N // _RB,),
        in_specs=[pl.BlockSpec((NC, _RB, DH), lambda i: (0, i, 0)),
                  pl.BlockSpec((NC, _RB, DEGW), lambda i: (0, i, 0)),
                  pl.BlockSpec((_RB, D), lambda i: (i, 0))],
        out_specs=pl.BlockSpec((_RB, D), lambda i: (i, 0)),
        out_shape=jax.ShapeDtypeStruct((N, D), _F32),
    )(acc2, deg, hr)


def kernel(x, edge_index, W1_l, b1_l, W1_r, W2_l, b2_l, W2_r):
    src = edge_index[0].astype(jnp.int32).reshape(NS, NCHUNK, CHUNK)
    dst = edge_index[1].astype(jnp.int32).reshape(NS, NCHUNK, CHUNK)
    ones = jnp.zeros((CHUNK, DEGW), _F32).at[:, 0].set(1.0)
    zacc = jnp.zeros((NP, DH), _F32)
    zdeg = jnp.zeros((NP, DEGW), _F32)
    b1 = b1_l.reshape(1, D)
    b2 = b2_l.reshape(1, D)

    xl, xr = _tc_pre(x, W1_l, W1_r, b1)
    acc1, deg = _SC_AGG_DEG(xl, src, dst, ones, zacc, zdeg)
    hl, hr = _tc_mid(acc1, deg, xr, W2_l, W2_r, b2)
    acc2 = _SC_AGG(hl, src, dst, zacc)
    return _tc_post(acc2, deg, hr)
